# Initial kernel scaffold; baseline (speedup 1.0000x reference)
import jax
import jax.numpy as jnp
from jax.experimental import pallas as pl


def kernel(wi, cond, W1, b1, W2, b2):
    raise NotImplementedError("write your pallas kernel here")


# speedup vs baseline: 3.1194x; 3.1194x over previous
import functools
import math

import jax
import jax.numpy as jnp
from jax.experimental import pallas as pl
from jax.experimental.pallas import tpu as pltpu

_K = 2
_NUM_ENC = 5
_TWO_PI = 2.0 * math.pi
_INV_PI = 1.0 / math.pi


def _gmm_body(wi_ref, cond_ref, w1a_ref, w1s_ref, w1c_ref, b1_ref,
              w2t_ref, b2_ref, out_ref, *, freqs):
    cond_t = cond_ref[...]
    wx = wi_ref[0:1, :]
    wy = wi_ref[1:2, :]

    del freqs
    f_col = 2.0 ** jax.lax.broadcasted_iota(
        jnp.int32, (_NUM_ENC, 1), 0).astype(jnp.float32)
    ux = f_col * cond_t[8:9, :]
    uy = f_col * cond_t[9:10, :]
    u = jnp.concatenate([ux, uy], axis=0)

    hpre = (jnp.dot(w1a_ref[...], cond_t, preferred_element_type=jnp.float32)
            + jnp.dot(w1s_ref[...], jnp.sin(u), preferred_element_type=jnp.float32)
            + jnp.dot(w1c_ref[...], jnp.cos(u), preferred_element_type=jnp.float32)
            + b1_ref[...])
    h = jnp.maximum(hpre, 0.0)
    ret = jnp.dot(w2t_ref[...], h, preferred_element_type=jnp.float32) + b2_ref[...]

    pdf = jnp.zeros_like(wx)
    for k in range(_K):
        lsx = ret[4 + 2 * k:5 + 2 * k, :]
        lsy = ret[5 + 2 * k:6 + 2 * k, :]
        zx = (wx - ret[2 * k:2 * k + 1, :]) * jnp.exp(-lsx)
        zy = (wy - ret[2 * k + 1:2 * k + 2, :]) * jnp.exp(-lsy)
        g = jnp.exp(-0.5 * (zx * zx + zy * zy) - lsx - lsy)
        pdf = pdf + jnp.abs(ret[8 + k:9 + k, :]) * g
    pdf = pdf * (1.0 / _TWO_PI)

    w_uni = jnp.abs(ret[10:11, :])
    inside = jnp.where(wx * wx + wy * wy <= 1.0, _INV_PI, 0.0)
    pdf = pdf + w_uni * inside

    wsum = jnp.abs(ret[8:9, :]) + jnp.abs(ret[9:10, :]) + w_uni
    out_ref[...] = (pdf / jnp.maximum(wsum, 1e-12))[None]


@jax.jit
def kernel(wi, cond, W1, b1, W2, b2):
    B = wi.shape[0]
    N = 2048
    G = B // N

    freqs = tuple(2.0 ** i for i in range(_NUM_ENC))

    w1t = W1.T
    cond_cols = [22 + j for j in range(8)] + [0, 1]
    sin_cols = [2 + 4 * i for i in range(_NUM_ENC)] + [3 + 4 * i for i in range(_NUM_ENC)]
    cos_cols = [4 + 4 * i for i in range(_NUM_ENC)] + [5 + 4 * i for i in range(_NUM_ENC)]
    w1a = w1t[:, jnp.asarray(cond_cols)]
    w1s = w1t[:, jnp.asarray(sin_cols)]
    w1c = w1t[:, jnp.asarray(cos_cols)]

    wi_t = wi.T
    cond_t = cond.T
    b1c = b1.reshape(-1, 1)
    b2c = b2.reshape(-1, 1)
    w2t = W2.T

    col_spec = lambda r: pl.BlockSpec((r, N), lambda i: (0, i))
    full = lambda a: pl.BlockSpec(a.shape, lambda i: (0,) * a.ndim)

    pdf = pl.pallas_call(
        functools.partial(_gmm_body, freqs=freqs),
        grid=(G,),
        in_specs=[
            col_spec(2),
            col_spec(10),
            full(w1a), full(w1s), full(w1c), full(b1c),
            full(w2t), full(b2c),
        ],
        out_specs=pl.BlockSpec((1, 1, N), lambda i: (i, 0, 0)),
        out_shape=jax.ShapeDtypeStruct((G, 1, N), jnp.float32),
        compiler_params=pltpu.CompilerParams(
            dimension_semantics=("parallel",)),
    )(wi_t, cond_t, w1a, w1s, w1c, b1c, w2t, b2c)

    return (wi, pdf.reshape(B))
